# Initial kernel scaffold; baseline (speedup 1.0000x reference)
import jax
import jax.numpy as jnp
from jax.experimental import pallas as pl


def kernel(ids, table):
    raise NotImplementedError("write your pallas kernel here")


# speedup vs baseline: 2.0263x; 2.0263x over previous
import functools

import jax
import jax.numpy as jnp
from jax import lax
from jax.experimental import pallas as pl
from jax.experimental.pallas import tpu as pltpu
from jax.experimental.pallas import tpu_sc as plsc

_VOCAB = 1000000
_DIM = 32
_NC, _NS, _L = 2, 16, 16
_NW = _NC * _NS


def _compute_indices(v):
    lo = jnp.bitwise_and(v, jnp.int32(0xFFFF))
    hi = jnp.right_shift(v, jnp.int32(16))
    m = hi * jnp.int32(1024)
    q = (m.astype(jnp.float32) * jnp.float32(1.0 / 15625.0)).astype(jnp.int32)
    r = m - q * jnp.int32(15625)
    r = jnp.where(r < 0, r + jnp.int32(15625), r)
    r = jnp.where(r >= jnp.int32(15625), r - jnp.int32(15625), r)
    return r * jnp.int32(64), lo


def _make_sc_kernel(batch, chunk):
    assert batch % (_NW * chunk) == 0
    b_per_w = batch // _NW
    n_chunks = b_per_w // chunk
    mesh = plsc.VectorSubcoreMesh(core_axis_name="c", subcore_axis_name="s")

    @functools.partial(
        pl.kernel,
        out_type=jax.ShapeDtypeStruct((batch, _DIM), jnp.float32),
        mesh=mesh,
        compiler_params=pltpu.CompilerParams(use_tc_tiling_on_sc=False),
        scratch_types=[
            pltpu.VMEM((chunk,), jnp.int32),
            pltpu.VMEM((chunk,), jnp.int32),
            pltpu.VMEM((chunk,), jnp.int32),
            pltpu.VMEM((chunk, _DIM), jnp.float32),
            pltpu.VMEM((chunk, _DIM), jnp.float32),
            pltpu.SemaphoreType.DMA,
            pltpu.SemaphoreType.DMA,
        ],
    )
    def sc_kernel(ids_hbm, table_hbm, out_hbm,
                  ids_v, idx0_v, idx1_v, rows_a, rows_b, sem_a, sem_b):
        wid = lax.axis_index("s") * _NC + lax.axis_index("c")
        base = wid * b_per_w

        def chunk_body(c, carry):
            off = base + c * chunk
            pltpu.sync_copy(ids_hbm.at[pl.ds(off, chunk)], ids_v)

            def idx_body(i, carry2):
                v = ids_v[pl.ds(i * _L, _L)]
                i0, i1 = _compute_indices(v)
                idx0_v[pl.ds(i * _L, _L)] = i0
                idx1_v[pl.ds(i * _L, _L)] = i1
                return carry2

            lax.fori_loop(0, chunk // _L, idx_body, 0, unroll=4)

            cp_a = pltpu.async_copy(table_hbm.at[idx0_v], rows_a, sem_a)
            cp_b = pltpu.async_copy(table_hbm.at[idx1_v], rows_b, sem_b)
            cp_a.wait()
            cp_b.wait()

            def add_body(i, carry2):
                a0 = rows_a[i, pl.ds(0, _L)]
                b0 = rows_b[i, pl.ds(0, _L)]
                rows_a[i, pl.ds(0, _L)] = a0 + b0
                a1 = rows_a[i, pl.ds(_L, _L)]
                b1 = rows_b[i, pl.ds(_L, _L)]
                rows_a[i, pl.ds(_L, _L)] = a1 + b1
                return carry2

            lax.fori_loop(0, chunk, add_body, 0, unroll=4)

            pltpu.sync_copy(rows_a, out_hbm.at[pl.ds(off, chunk)])
            return carry

        lax.fori_loop(0, n_chunks, chunk_body, 0)

    return sc_kernel


def kernel(ids, table):
    batch = ids.shape[0] * ids.shape[1]
    flat_ids = ids.reshape(batch)
    sc = _make_sc_kernel(batch, chunk=1024)
    out = sc(flat_ids, table)
    return out.reshape(ids.shape[0], ids.shape[1], _DIM)


# speedup vs baseline: 2.3021x; 1.1361x over previous
import functools

import jax
import jax.numpy as jnp
from jax import lax
from jax.experimental import pallas as pl
from jax.experimental.pallas import tpu as pltpu
from jax.experimental.pallas import tpu_sc as plsc

_VOCAB = 1000000
_DIM = 32
_NC, _NS, _L = 2, 16, 16
_NW = _NC * _NS


def _compute_indices(v):
    lo = jnp.bitwise_and(v, jnp.int32(0xFFFF))
    hi = jnp.right_shift(v, jnp.int32(16))
    m = hi * jnp.int32(1024)
    q = (m.astype(jnp.float32) * jnp.float32(1.0 / 15625.0)).astype(jnp.int32)
    r = m - q * jnp.int32(15625)
    r = jnp.where(r < 0, r + jnp.int32(15625), r)
    r = jnp.where(r >= jnp.int32(15625), r - jnp.int32(15625), r)
    return r * jnp.int32(64), lo


def _make_sc_kernel(batch, chunk):
    assert batch % (_NW * chunk) == 0
    b_per_w = batch // _NW
    n_chunks = b_per_w // chunk
    mesh = plsc.VectorSubcoreMesh(core_axis_name="c", subcore_axis_name="s")

    @functools.partial(
        pl.kernel,
        out_type=jax.ShapeDtypeStruct((batch, _DIM), jnp.float32),
        mesh=mesh,
        compiler_params=pltpu.CompilerParams(use_tc_tiling_on_sc=False),
        scratch_types=[
            pltpu.VMEM((chunk,), jnp.int32),
            pltpu.VMEM((chunk,), jnp.int32),
            pltpu.VMEM((chunk,), jnp.int32),
            pltpu.VMEM((chunk, _DIM), jnp.float32),
            pltpu.VMEM((chunk, _DIM), jnp.float32),
            pltpu.SemaphoreType.DMA,
            pltpu.SemaphoreType.DMA,
        ],
    )
    def sc_kernel(ids_hbm, table_hbm, out_hbm,
                  ids_v, idx0_v, idx1_v, rows_a, rows_b, sem_a, sem_b):
        wid = lax.axis_index("s") * _NC + lax.axis_index("c")
        base = wid * b_per_w

        def chunk_body(c, carry):
            off = base + c * chunk
            pltpu.sync_copy(ids_hbm.at[pl.ds(off, chunk)], ids_v)

            def idx_body(i, carry2):
                v = ids_v[pl.ds(i * _L, _L)]
                i0, i1 = _compute_indices(v)
                idx0_v[pl.ds(i * _L, _L)] = i0
                idx1_v[pl.ds(i * _L, _L)] = i1
                return carry2

            lax.fori_loop(0, chunk // _L, idx_body, 0, unroll=4)

            cp_a = pltpu.async_copy(table_hbm.at[idx0_v], rows_a, sem_a)
            cp_a.wait()
            cp_b = pltpu.async_copy(table_hbm.at[idx1_v], rows_a, sem_b,
                                    add=True)
            cp_b.wait()

            pltpu.sync_copy(rows_a, out_hbm.at[pl.ds(off, chunk)])
            return carry

        lax.fori_loop(0, n_chunks, chunk_body, 0)

    return sc_kernel


def kernel(ids, table):
    batch = ids.shape[0] * ids.shape[1]
    flat_ids = ids.reshape(batch)
    sc = _make_sc_kernel(batch, chunk=1024)
    out = sc(flat_ids, table)
    return out.reshape(ids.shape[0], ids.shape[1], _DIM)


# speedup vs baseline: 2.3608x; 1.0255x over previous
import functools

import jax
import jax.numpy as jnp
from jax import lax
from jax.experimental import pallas as pl
from jax.experimental.pallas import tpu as pltpu
from jax.experimental.pallas import tpu_sc as plsc

_VOCAB = 1000000
_DIM = 32
_NC, _NS, _L = 2, 16, 16
_NW = _NC * _NS


def _compute_indices(v):
    lo = jnp.bitwise_and(v, jnp.int32(0xFFFF))
    hi = jnp.right_shift(v, jnp.int32(16))
    m = hi * jnp.int32(1024)
    q = (m.astype(jnp.float32) * jnp.float32(1.0 / 15625.0)).astype(jnp.int32)
    r = m - q * jnp.int32(15625)
    r = jnp.where(r < 0, r + jnp.int32(15625), r)
    r = jnp.where(r >= jnp.int32(15625), r - jnp.int32(15625), r)
    return r * jnp.int32(64), lo


def _make_sc_kernel(batch, chunk, n_slots=3):
    assert batch % (_NW * chunk) == 0
    b_per_w = batch // _NW
    n_chunks = b_per_w // chunk
    mesh = plsc.VectorSubcoreMesh(core_axis_name="c", subcore_axis_name="s")

    @functools.partial(
        pl.kernel,
        out_type=jax.ShapeDtypeStruct((batch, _DIM), jnp.float32),
        mesh=mesh,
        compiler_params=pltpu.CompilerParams(use_tc_tiling_on_sc=False),
        scratch_types=[
            pltpu.VMEM((b_per_w,), jnp.int32),
            pltpu.VMEM((n_chunks, chunk), jnp.int32),
            pltpu.VMEM((n_chunks, chunk), jnp.int32),
            [pltpu.VMEM((chunk, _DIM), jnp.float32)] * n_slots,
            [pltpu.SemaphoreType.DMA] * n_slots,
            [pltpu.SemaphoreType.DMA] * n_slots,
        ],
    )
    def sc_kernel(ids_hbm, table_hbm, out_hbm,
                  ids_v, idx0_v, idx1_v, rows, sem_g, sem_o):
        wid = lax.axis_index("s") * _NC + lax.axis_index("c")
        base = wid * b_per_w

        pltpu.sync_copy(ids_hbm.at[pl.ds(base, b_per_w)], ids_v)

        def idx_body(i, carry):
            c = i // (chunk // _L)
            j = i % (chunk // _L)
            v = ids_v[pl.ds(i * _L, _L)]
            i0, i1 = _compute_indices(v)
            idx0_v[c, pl.ds(j * _L, _L)] = i0
            idx1_v[c, pl.ds(j * _L, _L)] = i1
            return carry

        lax.fori_loop(0, b_per_w // _L, idx_body, 0, unroll=8)

        cp_a = [None] * n_slots
        cp_b = [None] * n_slots
        cp_o = [None] * n_slots

        def start_a(c):
            s = c % n_slots
            if cp_o[s] is not None:
                cp_o[s].wait()
            cp_a[s] = pltpu.async_copy(table_hbm.at[idx0_v.at[c]], rows[s],
                                       sem_g[s])

        def a_to_b(c):
            s = c % n_slots
            cp_a[s].wait()
            cp_b[s] = pltpu.async_copy(table_hbm.at[idx1_v.at[c]], rows[s],
                                       sem_g[s], add=True)

        def b_to_o(c):
            s = c % n_slots
            cp_b[s].wait()
            cp_o[s] = pltpu.async_copy(
                rows[s], out_hbm.at[pl.ds(base + c * chunk, chunk)], sem_o[s])

        for c in range(n_chunks + 2):
            if c < n_chunks:
                start_a(c)
            if 1 <= c < n_chunks + 1:
                a_to_b(c - 1)
            if 2 <= c:
                b_to_o(c - 2)
        for s in range(n_slots):
            cp_o[s].wait()

    return sc_kernel


def kernel(ids, table):
    batch = ids.shape[0] * ids.shape[1]
    flat_ids = ids.reshape(batch)
    sc = _make_sc_kernel(batch, chunk=832)
    out = sc(flat_ids, table)
    return out.reshape(ids.shape[0], ids.shape[1], _DIM)
